# baseline (speedup 1.0000x reference)
import jax
import jax.numpy as jnp
from jax import lax
from jax.experimental import pallas as pl
from jax.experimental.pallas import tpu as pltpu
from jax.experimental.pallas import tpu_sc as plsc

_TS = 512
_B = 64
_NC, _NS, _L = 2, 16, 16
_NW = _NC * _NS
_ROWS = _TS // _NW
_CHUNK = _ROWS * _TS


def _sc_body(out_hbm, chunk_v, sem):
    wid = lax.axis_index("s") * _NC + lax.axis_index("c")
    row0 = wid * _ROWS

    def row_body(r, _):
        d = row0 + r
        limit = _TS - d

        m1 = (d - _TS // 4) >> 31
        m2 = (d - _TS // 2) >> 31

        def col_body(c, _):
            s = lax.iota(jnp.int32, _L) + c * _L
            lt = (s - limit) >> 31
            even = ((s & 1) - 1) >> 31
            mod4 = ((s & 3) - 1) >> 31
            stride = m1 | (m2 & even) | mod4
            bits = lt & stride & jnp.int32(0x3F800000)
            chunk_v[pl.ds(r * _TS + c * _L, _L)] = lax.bitcast_convert_type(
                bits, jnp.float32
            )
            return 0

        return lax.fori_loop(0, _TS // _L, col_body, 0)

    lax.fori_loop(0, _ROWS, row_body, 0)

    def fire(b, _):
        pltpu.async_copy(
            chunk_v, out_hbm.at[pl.ds(b * _TS * _TS + row0 * _TS, _CHUNK)], sem
        )
        return 0

    lax.fori_loop(0, _B, fire, 0)

    def drain(b, _):
        pltpu.make_async_copy(
            chunk_v, out_hbm.at[pl.ds(b * _TS * _TS + row0 * _TS, _CHUNK)], sem
        ).wait()
        return 0

    lax.fori_loop(0, _B, drain, 0)


def kernel(start, end, actionness):
    f = pl.kernel(
        _sc_body,
        out_type=jax.ShapeDtypeStruct((_B * _TS * _TS,), jnp.float32),
        mesh=plsc.VectorSubcoreMesh(core_axis_name="c", subcore_axis_name="s"),
        scratch_types=[
            pltpu.VMEM((_CHUNK,), jnp.float32),
            pltpu.SemaphoreType.DMA,
        ],
    )
    return f().reshape(_B, _TS, _TS)
